# baseline (speedup 1.0000x reference)
import functools

import jax
import jax.numpy as jnp
from jax import lax
from jax.experimental import pallas as pl
from jax.experimental.pallas import tpu as pltpu
from jax.experimental.pallas import tpu_sc as plsc

VOCAB = 1000000
DIM = 32
B = 4096
L = 200

_INFO = plsc.get_sparse_core_info()
NC = _INFO.num_cores
NS = _INFO.num_subcores
NW = NC * NS
BW = B // NW
NBUF = 2
DH = DIM // 8


def _make_sc_gather():
  mesh = plsc.VectorSubcoreMesh(core_axis_name="c", subcore_axis_name="s")

  @functools.partial(
      pl.kernel,
      mesh=mesh,
      compiler_params=pltpu.CompilerParams(use_tc_tiling_on_sc=False,
                                           needs_layout_passes=False),
      out_type=jax.ShapeDtypeStruct((L, DH, B // 128, 8, 128), jnp.float32),
      scratch_types=[
          pltpu.VMEM((BW, L), jnp.int32),
          pltpu.VMEM((NBUF, BW), jnp.int32),
          pltpu.VMEM((NBUF, BW), jnp.int32),
          pltpu.VMEM((NBUF, BW, 128), jnp.float32),
          pltpu.VMEM((NBUF, DH, 8, 128), jnp.float32),
          pltpu.SemaphoreType.DMA((NBUF,)),
          pltpu.SemaphoreType.DMA((NBUF,)),
      ],
  )
  def gather_kernel(idx_hbm, table4_hbm, out5_hbm, idx_v, widx_v, qoff_v,
                    wide_v, tbuf, gsem, osem):
    wid = lax.axis_index("s") * NC + lax.axis_index("c")
    b0 = wid * BW

    pltpu.sync_copy(idx_hbm.at[pl.ds(b0, BW)], idx_v)

    iota = lax.iota(jnp.int32, 16)

    def prep_and_fire_gather(l, s):
      lv = jnp.zeros((16,), jnp.int32) + l
      for gb in range(8):
        v = plsc.load_gather(idx_v, [iota + gb * 16, lv])
        widx_v[s, pl.ds(gb * 16, 16)] = lax.shift_right_logical(v, 2)
        qoff_v[s, pl.ds(gb * 16, 16)] = lax.shift_left(v & 3, 5)
      pltpu.async_copy(table4_hbm.at[widx_v.at[s]], wide_v.at[s], gsem.at[s])

    def wait_gather(s):
      pltpu.make_async_copy(table4_hbm.at[widx_v.at[s]], wide_v.at[s],
                            gsem.at[s]).wait()

    zeros = jnp.zeros((16,), jnp.int32)

    def transpose(s):
      for gb in range(8):
        q = qoff_v[s, pl.ds(gb * 16, 16)]
        addr = q + (iota + gb * 16) * 128
        for w in range(DIM):
          v = plsc.load_gather(wide_v.at[s], [zeros, addr + w])
          tbuf[s, w // 8, w % 8, pl.ds(gb * 16, 16)] = v

    def fire_out(l, s):
      for dh in range(DH):
        pltpu.async_copy(tbuf.at[s].at[dh], out5_hbm.at[l].at[dh].at[wid],
                         osem.at[s])

    def wait_out(l, s):
      for dh in range(DH):
        pltpu.make_async_copy(tbuf.at[s].at[dh], out5_hbm.at[l].at[dh].at[wid],
                              osem.at[s]).wait()

    for s in range(NBUF):
      prep_and_fire_gather(s, s)

    def body(l0, carry):
      for s in range(NBUF):
        l = l0 + s
        wait_gather(s)

        @pl.when(l >= NBUF)
        def _():
          wait_out(l, s)

        transpose(s)
        fire_out(l, s)

        @pl.when(l + NBUF < L)
        def _():
          prep_and_fire_gather(l + NBUF, s)
      return carry

    lax.fori_loop(0, L // NBUF, lambda i, c: body(i * NBUF, c), 0,
                  unroll=False)

    for s in range(NBUF):
      wait_out(L - NBUF + s, s)

  return gather_kernel


_sc_gather = _make_sc_gather()


@jax.jit
def kernel(inputs, table):
  out5 = _sc_gather(inputs.astype(jnp.int32), table.reshape(VOCAB // 4, 128))
  return out5.transpose(2, 4, 0, 1, 3).reshape(B, L, DIM)


# speedup vs baseline: 1.0397x; 1.0397x over previous
import functools

import jax
import jax.numpy as jnp
from jax import lax
from jax.experimental import pallas as pl
from jax.experimental.pallas import tpu as pltpu
from jax.experimental.pallas import tpu_sc as plsc

VOCAB = 1000000
DIM = 32
B = 4096
L = 200

_INFO = plsc.get_sparse_core_info()
NC = _INFO.num_cores
NS = _INFO.num_subcores
NW = NC * NS
BW = B // NW
NBUF = 2
DH = DIM // 8


def _make_sc_gather():
  mesh = plsc.VectorSubcoreMesh(core_axis_name="c", subcore_axis_name="s")

  @functools.partial(
      pl.kernel,
      mesh=mesh,
      compiler_params=pltpu.CompilerParams(use_tc_tiling_on_sc=False,
                                           needs_layout_passes=False),
      out_type=jax.ShapeDtypeStruct((L, DH, B // 128, 8, 128), jnp.float32),
      scratch_types=[
          pltpu.VMEM((BW, L), jnp.int32),
          pltpu.VMEM((NBUF, BW), jnp.int32),
          pltpu.VMEM((NBUF, BW), jnp.int32),
          pltpu.VMEM((NBUF, BW, 128), jnp.float32),
          pltpu.VMEM((NBUF, DH, 8, 128), jnp.float32),
          pltpu.VMEM((DIM, 16), jnp.int32),
          pltpu.SemaphoreType.DMA((NBUF,)),
          pltpu.SemaphoreType.DMA((NBUF,)),
      ],
  )
  def gather_kernel(idx_hbm, table4_hbm, out5_hbm, idx_v, widx_v, qoff_v,
                    wide_v, tbuf, dtab, gsem, osem):
    wid = lax.axis_index("s") * NC + lax.axis_index("c")
    b0 = wid * BW

    pltpu.sync_copy(idx_hbm.at[pl.ds(b0, BW)], idx_v)

    iota = lax.iota(jnp.int32, 16)

    def prep_and_fire_gather(l, s):
      lv = jnp.zeros((16,), jnp.int32) + l
      for gb in range(8):
        v = plsc.load_gather(idx_v, [iota + gb * 16, lv])
        widx_v[s, pl.ds(gb * 16, 16)] = lax.shift_right_logical(v, 2)
        qoff_v[s, pl.ds(gb * 16, 16)] = lax.shift_left(v & 3, 5)
      pltpu.async_copy(table4_hbm.at[widx_v.at[s]], wide_v.at[s], gsem.at[s])

    def wait_gather(s):
      pltpu.make_async_copy(table4_hbm.at[widx_v.at[s]], wide_v.at[s],
                            gsem.at[s]).wait()

    zeros = jnp.zeros((16,), jnp.int32)

    for w0 in range(DIM):
      wv = (iota + w0) & 31
      dtab[w0] = (lax.shift_left(lax.shift_right_logical(wv, 3), 10)
                  | lax.shift_left(wv & 7, 7))

    def transpose(s):
      for gb in range(8):
        q = qoff_v[s, pl.ds(gb * 16, 16)]
        bvec = iota + gb * 16
        srcbase = q + bvec * 128
        for w0 in range(DIM):
          dv = dtab[w0]
          wv = (lax.shift_left(lax.shift_right_logical(dv, 10), 3)
                | (lax.shift_right_logical(dv, 7) & 7))
          v = plsc.load_gather(wide_v.at[s], [zeros, srcbase + wv])
          plsc.store_scatter(tbuf.at[s], [zeros, zeros, dv + bvec], v)

    def fire_out(l, s):
      for dh in range(DH):
        pltpu.async_copy(tbuf.at[s].at[dh], out5_hbm.at[l].at[dh].at[wid],
                         osem.at[s])

    def wait_out(l, s):
      for dh in range(DH):
        pltpu.make_async_copy(tbuf.at[s].at[dh], out5_hbm.at[l].at[dh].at[wid],
                              osem.at[s]).wait()

    for s in range(NBUF):
      prep_and_fire_gather(s, s)

    def body(l0, carry):
      for s in range(NBUF):
        l = l0 + s
        wait_gather(s)

        @pl.when(l >= NBUF)
        def _():
          wait_out(l, s)

        transpose(s)
        fire_out(l, s)

        @pl.when(l + NBUF < L)
        def _():
          prep_and_fire_gather(l + NBUF, s)
      return carry

    lax.fori_loop(0, L // NBUF, lambda i, c: body(i * NBUF, c), 0,
                  unroll=False)

    for s in range(NBUF):
      wait_out(L - NBUF + s, s)

  return gather_kernel


_sc_gather = _make_sc_gather()


@jax.jit
def kernel(inputs, table):
  out5 = _sc_gather(inputs.astype(jnp.int32), table.reshape(VOCAB // 4, 128))
  return out5.transpose(2, 4, 0, 1, 3).reshape(B, L, DIM)


# speedup vs baseline: 1.1712x; 1.1265x over previous
import functools

import jax
import jax.numpy as jnp
from jax import lax
from jax.experimental import pallas as pl
from jax.experimental.pallas import tpu as pltpu
from jax.experimental.pallas import tpu_sc as plsc

VOCAB = 1000000
DIM = 32
B = 4096
L = 200

_INFO = plsc.get_sparse_core_info()
NC = _INFO.num_cores
NS = _INFO.num_subcores
NW = NC * NS
BW = B // NW

CB = 4
SEGS = ((0, 104), (104, 96))
NCHUNK = BW // CB
NBUF = 2


def _make_sc_gather():
  mesh = plsc.VectorSubcoreMesh(core_axis_name="c", subcore_axis_name="s")

  @functools.partial(
      pl.kernel,
      mesh=mesh,
      compiler_params=pltpu.CompilerParams(use_tc_tiling_on_sc=False),
      out_type=jax.ShapeDtypeStruct((B, L, DIM), jnp.float32),
      scratch_types=[
          pltpu.VMEM((BW, L), jnp.int32),
          pltpu.VMEM((NBUF, CB, L, DIM), jnp.float32),
          pltpu.SemaphoreType.DMA((NBUF,)),
          pltpu.SemaphoreType.DMA((NBUF,)),
      ],
  )
  def gather_kernel(idx_hbm, table_hbm, out_hbm, idx_v, rows_v, gsem, osem):
    wid = lax.axis_index("s") * NC + lax.axis_index("c")
    b_base = wid * BW

    pltpu.sync_copy(idx_hbm.at[pl.ds(b_base, BW)], idx_v)

    def each_stream(g, slot, fn):
      for r in range(CB):
        for off, seg in SEGS:
          idx_slice = idx_v.at[g * CB + r].at[pl.ds(off, seg)]
          dst = rows_v.at[slot].at[r].at[pl.ds(off, seg)]
          fn(table_hbm.at[idx_slice], dst, gsem.at[slot])

    def fire_gathers(g, slot):
      each_stream(g, slot, pltpu.async_copy)

    def wait_gathers(g, slot):
      each_stream(g, slot, lambda s, d, m: pltpu.make_async_copy(s, d, m).wait())

    def fire_out(g, slot):
      pltpu.async_copy(rows_v.at[slot],
                       out_hbm.at[pl.ds(b_base + g * CB, CB)], osem.at[slot])

    def wait_out(g, slot):
      pltpu.make_async_copy(rows_v.at[slot],
                            out_hbm.at[pl.ds(b_base + g * CB, CB)],
                            osem.at[slot]).wait()

    for b in range(NBUF):
      fire_gathers(b, b)

    def body(g0, carry):
      for b in range(NBUF):
        g = g0 + b
        wait_gathers(g, b)
        fire_out(g, b)
        ng = g + NBUF

        @pl.when(ng < NCHUNK)
        def _():
          wait_out(g, b)
          fire_gathers(ng, b)

        @pl.when(ng >= NCHUNK)
        def _():
          wait_out(g, b)
      return carry

    lax.fori_loop(0, NCHUNK // NBUF, lambda i, c: body(i * NBUF, c), 0,
                  unroll=False)

  return gather_kernel


_sc_gather = _make_sc_gather()


@jax.jit
def kernel(inputs, table):
  t128 = lax.optimization_barrier(table.reshape(VOCAB // 4, 128))
  out = _sc_gather(inputs.astype(jnp.int32), t128.reshape(VOCAB, DIM))
  o = lax.optimization_barrier(out.reshape(B * L * DIM // 1024, 8, 128))
  return o.reshape(B, L, DIM)
